# Initial kernel scaffold; baseline (speedup 1.0000x reference)
import jax
import jax.numpy as jnp
from jax.experimental import pallas as pl


def kernel(x, edge_index, edge_weight, kernel, bias):
    raise NotImplementedError("write your pallas kernel here")


# speedup vs baseline: 5.0133x; 5.0133x over previous
import functools

import jax
import jax.numpy as jnp
from jax import lax
from jax.experimental import pallas as pl
from jax.experimental.pallas import tpu as pltpu
from jax.experimental.pallas import tpu_sc as plsc

N_NODES = 10000
N_EDGES = 320000
D = 128

NC = 2
NS = 16
NW = NC * NS
L = 16

CH = 128
N_CHUNKS = N_EDGES // CH
BASE_CHUNKS = N_CHUNKS // NW
REM_CHUNKS = N_CHUNKS % NW
N_PAD = 10240
ROWS_PER_SUB = N_PAD // NS

_mesh = plsc.VectorSubcoreMesh(core_axis_name="c", subcore_axis_name="s")


@functools.partial(
    pl.kernel,
    out_type=jax.ShapeDtypeStruct((NC, N_PAD, D), jnp.float32),
    mesh=_mesh,
    scratch_types=[
        pltpu.VMEM((CH,), jnp.int32),
        pltpu.VMEM((CH,), jnp.int32),
        pltpu.VMEM((CH,), jnp.float32),
        pltpu.VMEM((CH, D), jnp.float32),
        pltpu.VMEM_SHARED((N_PAD, D), jnp.float32),
        pltpu.SemaphoreType.DMA,
    ],
)
def _sc_aggregate(x_hbm, col_hbm, row_hbm, w_hbm, zeros_hbm, out_hbm,
                  col_v, row_v, w_v, rows_v, acc_sh, sem):
    c = lax.axis_index("c")
    s = lax.axis_index("s")
    wid = s * NC + c

    pltpu.sync_copy(
        zeros_hbm.at[pl.ds(s * ROWS_PER_SUB, ROWS_PER_SUB)],
        acc_sh.at[pl.ds(s * ROWS_PER_SUB, ROWS_PER_SUB)],
    )
    plsc.subcore_barrier()

    n_chunks = BASE_CHUNKS + jnp.where(wid < REM_CHUNKS, 1, 0)

    def chunk_body(i, carry):
        chunk = wid + i * NW
        off = chunk * CH
        pltpu.sync_copy(col_hbm.at[pl.ds(off, CH)], col_v)
        pltpu.sync_copy(row_hbm.at[pl.ds(off, CH)], row_v)
        pltpu.sync_copy(w_hbm.at[pl.ds(off, CH)], w_v)
        pltpu.async_copy(x_hbm.at[col_v], rows_v, sem).wait()

        def scale_group(e16, carry2):
            w16 = w_v[pl.ds(e16 * L, L)]
            for j in range(L):
                e = e16 * L + j
                wj = w16[j]
                for g in range(D // L):
                    rows_v[e, pl.ds(g * L, L)] = rows_v[e, pl.ds(g * L, L)] * wj
            return carry2

        lax.fori_loop(0, CH // L, scale_group, 0)

        pltpu.sync_copy(rows_v, acc_sh.at[row_v], add=True)
        return carry

    lax.fori_loop(0, n_chunks, chunk_body, 0)

    plsc.subcore_barrier()
    pltpu.sync_copy(
        acc_sh.at[pl.ds(s * ROWS_PER_SUB, ROWS_PER_SUB)],
        out_hbm.at[c, pl.ds(s * ROWS_PER_SUB, ROWS_PER_SUB)],
    )


ROWS_BLK = 1000


def _tc_finish(p_ref, w_ref, b_ref, o_ref):
    agg = p_ref[0] + p_ref[1]
    y = jnp.dot(agg, w_ref[...], preferred_element_type=jnp.float32)
    o_ref[...] = jnp.maximum(y + b_ref[...], 0.0)


def kernel(x, edge_index, edge_weight, kernel, bias):
    row = edge_index[0].astype(jnp.int32)
    col = edge_index[1].astype(jnp.int32)
    zeros = jnp.zeros((N_PAD, D), jnp.float32)
    partials = _sc_aggregate(x, col, row, edge_weight, zeros)
    out = pl.pallas_call(
        _tc_finish,
        grid=(N_NODES // ROWS_BLK,),
        in_specs=[
            pl.BlockSpec((NC, ROWS_BLK, D), lambda i: (0, i, 0)),
            pl.BlockSpec((D, D), lambda i: (0, 0)),
            pl.BlockSpec((1, D), lambda i: (0, 0)),
        ],
        out_specs=pl.BlockSpec((ROWS_BLK, D), lambda i: (i, 0)),
        out_shape=jax.ShapeDtypeStruct((N_NODES, D), jnp.float32),
    )(partials, kernel, bias.reshape(1, D))
    return out
